# Initial kernel scaffold; baseline (speedup 1.0000x reference)
import jax
import jax.numpy as jnp
from jax.experimental import pallas as pl


def kernel(x, edge_index, W1l, b1, W1r, W2l, b2, W2r):
    raise NotImplementedError("write your pallas kernel here")


# speedup vs baseline: 5.6122x; 5.6122x over previous
import functools

import jax
import jax.numpy as jnp
from jax import lax
from jax.experimental import pallas as pl
from jax.experimental.pallas import tpu as pltpu
from jax.experimental.pallas import tpu_sc as plsc

N = 10000
E = 320000
D_IN = 128
D_HID = 128
N_CLASS = 64

NC = 2
NS = 16
G = 128
CHUNKS = 160
EPAD = NS * CHUNKS * G
NPAD = 10112
RPT = NPAD // NS
W1 = 72
XTRA = W1 - (D_IN - W1) - 1
W2 = 32

_mesh = plsc.VectorSubcoreMesh(core_axis_name="c", subcore_axis_name="s")


def _seg_sum_body(w, table_l, table_r, src2d, dst2d, zeros, out, srcbuf,
                  dstbuf, rows_a, rows_b, agg, sem_a, sem_b):
    c = lax.axis_index("c")
    s = lax.axis_index("s")
    pltpu.sync_copy(zeros, agg.at[pl.ds(s * RPT, RPT)])
    pltpu.sync_copy(src2d.at[pl.ds(s * CHUNKS, CHUNKS)], srcbuf)
    pltpu.sync_copy(dst2d.at[pl.ds(s * CHUNKS, CHUNKS)], dstbuf)
    plsc.subcore_barrier()

    def run(table):
        pltpu.async_copy(table.at[srcbuf.at[0]], rows_a, sem_a)

        @pl.loop(0, CHUNKS, step=2)
        def _(j):
            pltpu.async_copy(table.at[srcbuf.at[j + 1]], rows_b, sem_b)
            pltpu.make_async_copy(table.at[srcbuf.at[j]], rows_a, sem_a).wait()
            pltpu.sync_copy(rows_a, agg.at[dstbuf.at[j]], add=True)

            @pl.when(j + 2 < CHUNKS)
            def _():
                pltpu.async_copy(table.at[srcbuf.at[j + 2]], rows_a, sem_a)

            pltpu.make_async_copy(table.at[srcbuf.at[j + 1]], rows_b,
                                  sem_b).wait()
            pltpu.sync_copy(rows_b, agg.at[dstbuf.at[j + 1]], add=True)

    @pl.when(c == 0)
    def _():
        run(table_l)

    @pl.when(c == 1)
    def _():
        run(table_r)

    plsc.subcore_barrier()
    pltpu.sync_copy(agg.at[pl.ds(s * RPT, RPT)],
                    out.at[c, pl.ds(s * RPT, RPT)])


def _make_seg_sum(w):
    return pl.kernel(
        functools.partial(_seg_sum_body, w),
        out_type=jax.ShapeDtypeStruct((NC, NPAD, w), jnp.float32),
        mesh=_mesh,
        scratch_types=[
            pltpu.VMEM((CHUNKS, G), jnp.int32),
            pltpu.VMEM((CHUNKS, G), jnp.int32),
            pltpu.VMEM((G, w), jnp.float32),
            pltpu.VMEM((G, w), jnp.float32),
            pltpu.VMEM_SHARED((NPAD, w), jnp.float32),
            pltpu.SemaphoreType.DMA,
            pltpu.SemaphoreType.DMA,
        ],
        compiler_params=pltpu.CompilerParams(use_tc_tiling_on_sc=False),
    )


_seg_sum_1 = _make_seg_sum(W1)
_seg_sum_2 = _make_seg_sum(W2)


def _tc1_body(p_ref, x_ref, a1_ref, a2_ref, b1_ref, w1r_ref, w2l_ref,
              w2r_ref, b2_ref, tl_ref, tr_ref, zx_ref):
    pm = p_ref[0]
    pr = p_ref[1]
    cnt = jnp.maximum(pr[:, D_IN - W1:D_IN - W1 + 1], 1.0)
    h = jnp.dot(pm / cnt, a1_ref[...], preferred_element_type=jnp.float32)
    h = h + jnp.dot(pr[:, :D_IN - W1] / cnt, a2_ref[...],
                    preferred_element_type=jnp.float32)
    h = h + b1_ref[...]
    h = h + jnp.dot(x_ref[...], w1r_ref[...],
                    preferred_element_type=jnp.float32)
    h = jnp.maximum(h, 0.0)
    t = jnp.dot(h, w2l_ref[...], preferred_element_type=jnp.float32)
    tl_ref[...] = t[:, :W2]
    tr_ref[...] = t[:, W2:]
    z = jnp.dot(h, w2r_ref[...], preferred_element_type=jnp.float32)
    z = z + b2_ref[...]
    inv = jnp.broadcast_to(1.0 / cnt, (z.shape[0], N_CLASS))
    zx_ref[...] = jnp.concatenate([z, inv], axis=1)


def _tc2_body(q_ref, zx_ref, o_ref):
    qs = jnp.concatenate([q_ref[0], q_ref[1]], axis=1)
    z = zx_ref[:, :N_CLASS]
    inv = zx_ref[:, N_CLASS:N_CLASS + 1]
    o = z + qs * inv
    m = jnp.max(o, axis=1, keepdims=True)
    e = jnp.exp(o - m)
    o_ref[...] = (o - m) - jnp.log(jnp.sum(e, axis=1, keepdims=True))


_R1 = 632


def _tc1(p, xpad, a1, a2, b1, w1rt, w2lt, w2rt, b2):
    grid = (NPAD // _R1,)
    return pl.pallas_call(
        _tc1_body,
        grid=grid,
        in_specs=[
            pl.BlockSpec((NC, _R1, W1), lambda i: (0, i, 0)),
            pl.BlockSpec((_R1, D_IN), lambda i: (i, 0)),
            pl.BlockSpec((W1, D_HID), lambda i: (0, 0)),
            pl.BlockSpec((D_IN - W1, D_HID), lambda i: (0, 0)),
            pl.BlockSpec((1, D_HID), lambda i: (0, 0)),
            pl.BlockSpec((D_IN, D_HID), lambda i: (0, 0)),
            pl.BlockSpec((D_HID, N_CLASS), lambda i: (0, 0)),
            pl.BlockSpec((D_HID, N_CLASS), lambda i: (0, 0)),
            pl.BlockSpec((1, N_CLASS), lambda i: (0, 0)),
        ],
        out_specs=[
            pl.BlockSpec((_R1, W2), lambda i: (i, 0)),
            pl.BlockSpec((_R1, W2), lambda i: (i, 0)),
            pl.BlockSpec((_R1, 2 * N_CLASS), lambda i: (i, 0)),
        ],
        out_shape=[
            jax.ShapeDtypeStruct((NPAD, W2), jnp.float32),
            jax.ShapeDtypeStruct((NPAD, W2), jnp.float32),
            jax.ShapeDtypeStruct((NPAD, 2 * N_CLASS), jnp.float32),
        ],
    )(p, xpad, a1, a2, b1, w1rt, w2lt, w2rt, b2)


def _tc2(q, zx):
    grid = (NPAD // _R1,)
    return pl.pallas_call(
        _tc2_body,
        grid=grid,
        in_specs=[
            pl.BlockSpec((NC, _R1, W2), lambda i: (0, i, 0)),
            pl.BlockSpec((_R1, 2 * N_CLASS), lambda i: (i, 0)),
        ],
        out_specs=pl.BlockSpec((_R1, N_CLASS), lambda i: (i, 0)),
        out_shape=jax.ShapeDtypeStruct((NPAD, N_CLASS), jnp.float32),
    )(q, zx)


@jax.jit
def kernel(x, edge_index, W1l, b1, W1r, W2l, b2, W2r):
    src = edge_index[0].astype(jnp.int32)
    dst = edge_index[1].astype(jnp.int32)
    pad = EPAD - E
    src2d = jnp.concatenate(
        [src, jnp.zeros((pad,), jnp.int32)]).reshape(NS * CHUNKS, G)
    dst2d = jnp.concatenate(
        [dst, jnp.full((pad,), N, jnp.int32)]).reshape(NS * CHUNKS, G)

    rows_pad = ((0, NPAD - N), (0, 0))
    xa_l = jnp.pad(x[:, :W1], rows_pad)
    xa_r = jnp.pad(
        jnp.concatenate(
            [x[:, W1:], jnp.ones((N, 1), jnp.float32),
             jnp.zeros((N, XTRA), jnp.float32)], axis=1), rows_pad)
    zeros1 = jnp.zeros((RPT, W1), jnp.float32)
    p = _seg_sum_1(xa_l, xa_r, src2d, dst2d, zeros1)

    xpad = jnp.pad(x, rows_pad)
    t_l, t_r, zx = _tc1(p, xpad, W1l.T[:W1], W1l.T[W1:], b1.reshape(1, -1),
                        W1r.T, W2l.T, W2r.T, b2.reshape(1, -1))

    zeros2 = jnp.zeros((RPT, W2), jnp.float32)
    q = _seg_sum_2(t_l, t_r, src2d, dst2d, zeros2)

    out = _tc2(q, zx)
    return out[:N]
